# baseline (speedup 1.0000x reference)
import functools

import jax
import jax.numpy as jnp
from jax import lax
from jax.experimental import pallas as pl
from jax.experimental.pallas import tpu as pltpu
from jax.experimental.pallas import tpu_sc as plsc

NUM_CORES = 2
NUM_SUBCORES = 16
NUM_WORKERS = NUM_CORES * NUM_SUBCORES
CHUNK = 128
GDEPTH = 6


def _make_emb(total, n_chunks, per_w, D, BBLK):
    mesh = plsc.VectorSubcoreMesh(core_axis_name="c", subcore_axis_name="s")
    S = 26
    CB, CI = D // 8, 8

    @functools.partial(
        pl.kernel,
        mesh=mesh,
        compiler_params=pltpu.CompilerParams(
            use_tc_tiling_on_sc=False, needs_layout_passes=False
        ),
        out_type=jax.ShapeDtypeStruct((S, CB, BBLK, CI, CHUNK), jnp.float32),
        scratch_types=[
            pltpu.VMEM((n_chunks, CHUNK), jnp.int32),
            pltpu.VMEM((GDEPTH, CHUNK, D), jnp.float32),
            pltpu.VMEM((2, CB, CI, CHUNK + 1), jnp.float32),
            pltpu.SemaphoreType.DMA,
            pltpu.SemaphoreType.DMA,
        ],
    )
    def emb(table_hbm, idx_hbm, out_hbm, idx_v, rows_v, tile_v, gsem, ssem):
        wid = lax.axis_index("s") * NUM_CORES + lax.axis_index("c")
        base = wid * n_chunks

        pltpu.sync_copy(idx_hbm.at[wid], idx_v)

        lanes = lax.iota(jnp.int32, 16)
        cb_vecs = [(lanes + p * 16) // 8 for p in range(2)]
        ci_vecs = [(lanes + p * 16) % 8 for p in range(2)]

        def gather(j):
            pltpu.async_copy(table_hbm.at[idx_v.at[j]], rows_v.at[j % GDEPTH], gsem)

        def wait_gather(j):
            pltpu.make_async_copy(
                table_hbm.at[idx_v.at[j]], rows_v.at[j % GDEPTH], gsem
            ).wait()

        def transpose(j):
            rslot, tslot = j % GDEPTH, j % 2

            def tbody(bi0, carry):
                for dbi in range(8):
                    bi = bi0 + dbi
                    bi_vec = jnp.full((16,), 0, jnp.int32) + bi
                    for p in range(2):
                        val = rows_v[rslot, bi, pl.ds(16 * p, 16)]
                        plsc.store_scatter(
                            tile_v.at[tslot], [cb_vecs[p], ci_vecs[p], bi_vec], val
                        )
                return carry

            lax.fori_loop(0, CHUNK // 8, lambda i, c: tbody(i * 8, c), 0)

        def store(j):
            t = base + j
            s, bb = t // BBLK, t % BBLK
            pltpu.async_copy(
                tile_v.at[j % 2, :, :, pl.ds(0, CHUNK)], out_hbm.at[s, :, bb], ssem
            )

        def wait_store(j):
            t = base + j
            s, bb = t // BBLK, t % BBLK
            pltpu.make_async_copy(
                tile_v.at[j % 2, :, :, pl.ds(0, CHUNK)], out_hbm.at[s, :, bb], ssem
            ).wait()

        for g in range(GDEPTH):
            gather(g)
        wait_gather(0)
        transpose(0)
        store(0)
        gather(GDEPTH)

        def body(j, carry):
            wait_gather(j)
            transpose(j)
            store(j)
            wait_store(j - 1)
            gather(j + GDEPTH)
            return carry

        lax.fori_loop(1, n_chunks - GDEPTH, body, 0)

        for j in range(n_chunks - GDEPTH, n_chunks):
            wait_gather(j)
            transpose(j)
            store(j)
            wait_store(j - 1)
        wait_store(n_chunks - 1)

    return emb


def _reshape_idx(xt):
    S, B = xt.shape
    GRID = 8
    BBLK = B // CHUNK

    def body(x_ref, o_ref):
        o_ref[...] = x_ref[...].reshape(S, -1, CHUNK)

    return pl.pallas_call(
        body,
        grid=(GRID,),
        in_specs=[pl.BlockSpec((S, B // GRID), lambda i: (0, i))],
        out_specs=pl.BlockSpec((S, BBLK // GRID, CHUNK), lambda i: (0, i, 0)),
        out_shape=jax.ShapeDtypeStruct((S, BBLK, CHUNK), jnp.int32),
    )(xt)


def _transpose_table(wt):
    D, V = wt.shape
    BK = 16384
    grid = (V + BK - 1) // BK

    def body(w_ref, o_ref):
        o_ref[...] = w_ref[...].T

    return pl.pallas_call(
        body,
        grid=(grid,),
        in_specs=[pl.BlockSpec((D, BK), lambda i: (0, i))],
        out_specs=pl.BlockSpec((BK, D), lambda i: (i, 0)),
        out_shape=jax.ShapeDtypeStruct((V, D), jnp.float32),
    )(wt)


def kernel(x, weight):
    B, S = x.shape
    V, D = weight.shape
    total = B * S
    per_w = total // NUM_WORKERS
    n_chunks = per_w // CHUNK
    BBLK = B // CHUNK
    xt = jnp.swapaxes(x.astype(jnp.int32), 0, 1)
    idx = _reshape_idx(xt).reshape(NUM_WORKERS, n_chunks, CHUNK)
    w_rows = _transpose_table(jnp.swapaxes(weight, 0, 1))
    out5 = _make_emb(total, n_chunks, per_w, D, BBLK)(w_rows, idx)
    return out5.transpose(2, 4, 0, 1, 3).reshape(B, S, D)


# speedup vs baseline: 1.9792x; 1.9792x over previous
import functools

import jax
import jax.numpy as jnp
from jax import lax
from jax.experimental import pallas as pl
from jax.experimental.pallas import tpu as pltpu
from jax.experimental.pallas import tpu_sc as plsc

NUM_CORES = 2
NUM_SUBCORES = 16
NUM_WORKERS = NUM_CORES * NUM_SUBCORES
CHUNK = 128
GDEPTH = 6


def _make_emb(total, n_chunks, per_w, D, BBLK):
    mesh = plsc.VectorSubcoreMesh(core_axis_name="c", subcore_axis_name="s")
    S = 26
    CB, CI = D // 8, 8

    @functools.partial(
        pl.kernel,
        mesh=mesh,
        compiler_params=pltpu.CompilerParams(
            use_tc_tiling_on_sc=False, needs_layout_passes=False
        ),
        out_type=jax.ShapeDtypeStruct((S, CB, BBLK, CI, CHUNK), jnp.float32),
        scratch_types=[
            pltpu.VMEM((n_chunks, CHUNK), jnp.int32),
            pltpu.VMEM((GDEPTH, CHUNK, D), jnp.float32),
            pltpu.VMEM((2, CB, CI, CHUNK + 1), jnp.float32),
            pltpu.SemaphoreType.DMA,
            pltpu.SemaphoreType.DMA,
        ],
    )
    def emb(table_hbm, idx_hbm, out_hbm, idx_v, rows_v, tile_v, gsem, ssem):
        wid = lax.axis_index("s") * NUM_CORES + lax.axis_index("c")
        base = wid * n_chunks

        pltpu.sync_copy(idx_hbm.at[wid], idx_v)

        lanes = lax.iota(jnp.int32, 16)
        cb_vecs = [(lanes + p * 16) // 8 for p in range(2)]
        ci_vecs = [(lanes + p * 16) % 8 for p in range(2)]

        def gather(j):
            pltpu.async_copy(table_hbm.at[idx_v.at[j]], rows_v.at[j % GDEPTH], gsem)

        def wait_gather(j):
            pltpu.make_async_copy(
                table_hbm.at[idx_v.at[j]], rows_v.at[j % GDEPTH], gsem
            ).wait()

        def transpose(j):
            rslot, tslot = j % GDEPTH, j % 2

            def tbody(bi0, carry):
                for dbi in range(8):
                    bi = bi0 + dbi
                    bi_vec = jnp.full((16,), 0, jnp.int32) + bi
                    for p in range(2):
                        val = rows_v[rslot, bi, pl.ds(16 * p, 16)]
                        plsc.store_scatter(
                            tile_v.at[tslot], [cb_vecs[p], ci_vecs[p], bi_vec], val
                        )
                return carry

            lax.fori_loop(0, CHUNK // 8, lambda i, c: tbody(i * 8, c), 0)

        def store(j):
            t = base + j
            s, bb = t // BBLK, t % BBLK
            pltpu.async_copy(
                tile_v.at[j % 2, :, :, pl.ds(0, CHUNK)], out_hbm.at[s, :, bb], ssem
            )

        def wait_store(j):
            t = base + j
            s, bb = t // BBLK, t % BBLK
            pltpu.make_async_copy(
                tile_v.at[j % 2, :, :, pl.ds(0, CHUNK)], out_hbm.at[s, :, bb], ssem
            ).wait()

        for g in range(GDEPTH):
            gather(g)
        wait_gather(0)
        transpose(0)
        store(0)
        gather(GDEPTH)

        def body(j, carry):
            wait_gather(j)
            transpose(j)
            store(j)
            wait_store(j - 1)
            gather(j + GDEPTH)
            return carry

        lax.fori_loop(1, n_chunks - GDEPTH, body, 0)

        for j in range(n_chunks - GDEPTH, n_chunks):
            wait_gather(j)
            transpose(j)
            store(j)
            wait_store(j - 1)
        wait_store(n_chunks - 1)

    return emb


def _reshape_idx(xt):
    S, B = xt.shape
    GRID = 8
    BBLK = B // CHUNK

    def body(x_ref, o_ref):
        o_ref[...] = (x_ref[...] * 4).reshape(S, -1, CHUNK)

    return pl.pallas_call(
        body,
        grid=(GRID,),
        in_specs=[pl.BlockSpec((S, B // GRID), lambda i: (0, i))],
        out_specs=pl.BlockSpec((S, BBLK // GRID, CHUNK), lambda i: (0, i, 0)),
        out_shape=jax.ShapeDtypeStruct((S, BBLK, CHUNK), jnp.int32),
    )(xt)


def _transpose_table(wt):
    D, V = wt.shape
    BK = 16384
    grid = (V + BK - 1) // BK

    def body(w_ref, o_ref):
        o_ref[:, 0:D] = w_ref[...].T

    return pl.pallas_call(
        body,
        grid=(grid,),
        in_specs=[pl.BlockSpec((D, BK), lambda i: (0, i))],
        out_specs=pl.BlockSpec((BK, 128), lambda i: (i, 0)),
        out_shape=jax.ShapeDtypeStruct((V, 128), jnp.float32),
    )(wt)


def kernel(x, weight):
    B, S = x.shape
    V, D = weight.shape
    total = B * S
    per_w = total // NUM_WORKERS
    n_chunks = per_w // CHUNK
    BBLK = B // CHUNK
    xt = jnp.swapaxes(x.astype(jnp.int32), 0, 1)
    idx = _reshape_idx(xt).reshape(NUM_WORKERS, n_chunks, CHUNK)
    w_rows = _transpose_table(jnp.swapaxes(weight, 0, 1)).reshape(4 * V, D)
    out5 = _make_emb(total, n_chunks, per_w, D, BBLK)(w_rows, idx)
    return out5.transpose(2, 4, 0, 1, 3).reshape(B, S, D)
